# Initial kernel scaffold; baseline (speedup 1.0000x reference)
import jax
import jax.numpy as jnp
from jax.experimental import pallas as pl


def kernel(x, edge_index, batch, W_rel1, b_rel1, W_root1, W_rel2, b_rel2, W_root2, W_rel3, b_rel3, W_root3, W_lin1, b_lin1, W_lin2, b_lin2):
    raise NotImplementedError("write your pallas kernel here")


# speedup vs baseline: 13.5089x; 13.5089x over previous
import functools

import jax
import jax.numpy as jnp
from jax import lax
from jax.experimental import pallas as pl
from jax.experimental.pallas import tpu as pltpu
from jax.experimental.pallas import tpu_sc as plsc

_N = 10000
_E = 320000
_H = 128
_G = 64
_XP = 16
_NC = 2
_NS = 16
_NT = _NC * _NS
_NP = 10240
_EPT = _E // _NT
_RPT = _NP // _NS
_K1 = 2000
_K2 = 200
_FPT = _N * _G // _NS
_ZC = 8000
_ZR = 128


def _sc1_body(xp_hbm, src_hbm, dst_hbm, batch_hbm, z16_hbm, zc_hbm, ones_hbm,
              agg1_out, cnt_out,
              sbuf, dbuf, fbuf, obuf, rows, bdbuf, zb16, zbc, acc1, accc, sem):
    cid = lax.axis_index("c")
    sid = lax.axis_index("s")
    r0 = sid * _RPT
    f0 = sid * _FPT
    pltpu.sync_copy(z16_hbm.at[pl.ds(r0, _RPT), :], zb16)
    pltpu.sync_copy(zb16, acc1.at[pl.ds(r0, _RPT), :])
    pltpu.sync_copy(zc_hbm.at[pl.ds(0, _ZC)], zbc)
    for k in range(_FPT // _ZC):
        pltpu.sync_copy(zbc, accc.at[pl.ds(f0 + k * _ZC, _ZC)])
    pltpu.sync_copy(ones_hbm, obuf)
    plsc.subcore_barrier()

    base = (sid * _NC + cid) * _EPT

    def chunk(c, carry):
        off = base + c * _K1
        pltpu.sync_copy(src_hbm.at[pl.ds(off, _K1)], sbuf)
        pltpu.sync_copy(dst_hbm.at[pl.ds(off, _K1)], dbuf)
        pltpu.async_copy(batch_hbm.at[dbuf], bdbuf, sem).wait()

        def inner(i, carry2):
            j = pl.multiple_of(i * 16, 16)
            bv = bdbuf[pl.ds(j, 16)]
            sv = sbuf[pl.ds(j, 16)]
            fbuf[pl.ds(j, 16)] = sv * _G + bv
            return carry2

        lax.fori_loop(0, _K1 // 16, inner, 0)
        pltpu.async_copy(xp_hbm.at[sbuf], rows, sem).wait()
        pltpu.sync_copy(rows, acc1.at[dbuf], add=True)
        pltpu.sync_copy(obuf, accc.at[fbuf], add=True)
        return carry

    lax.fori_loop(0, _EPT // _K1, chunk, 0)
    plsc.subcore_barrier()
    pltpu.sync_copy(acc1.at[pl.ds(r0, _RPT), :], zb16)
    pltpu.sync_copy(zb16, agg1_out.at[cid, pl.ds(r0, _RPT), :])
    for k in range(_FPT // _ZC):
        pltpu.sync_copy(accc.at[pl.ds(f0 + k * _ZC, _ZC)], zbc)
        pltpu.sync_copy(zbc, cnt_out.at[pl.ds(cid * _N * _G + f0 + k * _ZC, _ZC)])


def _sc1_call(xp, src, dst, batch, z16, zc, ones):
    mesh = plsc.VectorSubcoreMesh(core_axis_name="c", subcore_axis_name="s")
    f = pl.kernel(
        _sc1_body,
        out_type=[jax.ShapeDtypeStruct((_NC, _NP, _XP), jnp.float32),
                  jax.ShapeDtypeStruct((_NC * _N * _G,), jnp.float32)],
        mesh=mesh,
        scratch_types=[
            pltpu.VMEM((_K1,), jnp.int32),
            pltpu.VMEM((_K1,), jnp.int32),
            pltpu.VMEM((_K1,), jnp.int32),
            pltpu.VMEM((_K1,), jnp.float32),
            pltpu.VMEM((_K1, _XP), jnp.float32),
            pltpu.VMEM((_K1,), jnp.int32),
            pltpu.VMEM((_RPT, _XP), jnp.float32),
            pltpu.VMEM((_ZC,), jnp.float32),
            pltpu.VMEM_SHARED((_NP, _XP), jnp.float32),
            pltpu.VMEM_SHARED((_N * _G,), jnp.float32),
            pltpu.SemaphoreType.DMA,
        ],
        compiler_params=pltpu.CompilerParams(use_tc_tiling_on_sc=False),
    )
    return f(xp, src, dst, batch, z16, zc, ones)


def _sc2_body(h1_hbm, src_hbm, dst_hbm, z128_hbm, agg2_out,
              sbuf, dbuf, rows, acc, sem):
    cid = lax.axis_index("c")
    sid = lax.axis_index("s")
    r0 = sid * _RPT
    zb = rows.at[pl.ds(0, _ZR), :]
    pltpu.sync_copy(z128_hbm.at[pl.ds(0, _ZR), :], zb)
    for k in range(_RPT // _ZR):
        pltpu.sync_copy(zb, acc.at[pl.ds(r0 + k * _ZR, _ZR), :])
    plsc.subcore_barrier()

    base = (sid * _NC + cid) * _EPT

    def chunk(c, carry):
        off = base + c * _K2
        pltpu.sync_copy(src_hbm.at[pl.ds(off, _K2)], sbuf)
        pltpu.sync_copy(dst_hbm.at[pl.ds(off, _K2)], dbuf)
        pltpu.async_copy(h1_hbm.at[sbuf], rows, sem).wait()
        pltpu.sync_copy(rows, acc.at[dbuf], add=True)
        return carry

    lax.fori_loop(0, _EPT // _K2, chunk, 0)
    plsc.subcore_barrier()
    for k in range(_RPT // _ZR):
        pltpu.sync_copy(acc.at[pl.ds(r0 + k * _ZR, _ZR), :], zb)
        pltpu.sync_copy(zb, agg2_out.at[cid, pl.ds(r0 + k * _ZR, _ZR), :])


def _sc2_call(h1, src, dst, z128):
    mesh = plsc.VectorSubcoreMesh(core_axis_name="c", subcore_axis_name="s")
    f = pl.kernel(
        _sc2_body,
        out_type=jax.ShapeDtypeStruct((_NC, _NP, _H), jnp.float32),
        mesh=mesh,
        scratch_types=[
            pltpu.VMEM((_K2,), jnp.int32),
            pltpu.VMEM((_K2,), jnp.int32),
            pltpu.VMEM((_K2, _H), jnp.float32),
            pltpu.VMEM_SHARED((_NP, _H), jnp.float32),
            pltpu.SemaphoreType.DMA,
        ],
    )
    return f(h1, src, dst, z128)


_R = 1000
_NCH = _N // _R


def _tc1_body(a0, a1, xp, wr, wt, b, out):
    agg = a0[...] + a1[...]
    out[...] = jax.nn.relu(
        jnp.dot(agg, wr[...], preferred_element_type=jnp.float32)
        + jnp.dot(xp[...], wt[...], preferred_element_type=jnp.float32)
        + b[...])


def _tc1_call(a0, a1, xp, wr, wt, b):
    row = lambda i: (i, 0)
    fixed = lambda i: (0, 0)
    return pl.pallas_call(
        _tc1_body,
        grid=(_NCH,),
        in_specs=[
            pl.BlockSpec((_R, _XP), row),
            pl.BlockSpec((_R, _XP), row),
            pl.BlockSpec((_R, _XP), row),
            pl.BlockSpec((_XP, _H), fixed),
            pl.BlockSpec((_XP, _H), fixed),
            pl.BlockSpec((1, _H), fixed),
        ],
        out_specs=pl.BlockSpec((_R, _H), row),
        out_shape=jax.ShapeDtypeStruct((_N, _H), jnp.float32),
    )(a0, a1, xp, wr, wt, b)


def _tc2_body(a0, a1, h1, c0, c1, batch3, wr2, wt2, b2, wr3, wt3, b3,
              wl1, bl1, wl2, bl2, x1_out, x2_out, acc_s, acc_t, acc_n):
    i = pl.program_id(0)

    @pl.when(i == 0)
    def _init():
        acc_s[...] = jnp.zeros_like(acc_s)
        acc_t[...] = jnp.zeros_like(acc_t)
        acc_n[...] = jnp.zeros_like(acc_n)

    agg = a0[...] + a1[...]
    h2 = jax.nn.relu(
        jnp.dot(agg, wr2[...], preferred_element_type=jnp.float32)
        + jnp.dot(h1[...], wt2[...], preferred_element_type=jnp.float32)
        + b2[...])
    cc = c0[...] + c1[...]
    bv = batch3[0, 0, :]
    oh = (bv[:, None] == lax.broadcasted_iota(jnp.int32, (_R, _G), 1)
          ).astype(jnp.float32)
    dnum = (((0,), (0,)), ((), ()))
    acc_s[...] += lax.dot_general(cc, h2, dnum,
                                  preferred_element_type=jnp.float32)
    acc_t[...] += lax.dot_general(oh, h2, dnum,
                                  preferred_element_type=jnp.float32)
    acc_n[...] += jnp.sum(oh, axis=0, keepdims=True)

    @pl.when(i == _NCH - 1)
    def _fin():
        cnt = acc_n[0, :]
        pool_sum = (
            jnp.dot(acc_s[...], wr3[...], preferred_element_type=jnp.float32)
            + cnt[:, None] * b3[...]
            + jnp.dot(acc_t[...], wt3[...], preferred_element_type=jnp.float32))
        pooled = pool_sum / jnp.clip(cnt, 1.0, None)[:, None]
        x1_out[...] = jnp.dot(pooled, wl1[...],
                              preferred_element_type=jnp.float32) + bl1[...]
        x2_out[...] = jnp.dot(pooled, wl2[...],
                              preferred_element_type=jnp.float32) + bl2[...]


def _tc2_call(a0, a1, h1, c0, c1, batch3, wr2, wt2, b2, wr3, wt3, b3,
              wl1, bl1, wl2, bl2):
    row = lambda i: (i, 0)
    fixed = lambda i: (0, 0)
    return pl.pallas_call(
        _tc2_body,
        grid=(_NCH,),
        in_specs=[
            pl.BlockSpec((_R, _H), row),
            pl.BlockSpec((_R, _H), row),
            pl.BlockSpec((_R, _H), row),
            pl.BlockSpec((_R, _G), row),
            pl.BlockSpec((_R, _G), row),
            pl.BlockSpec((1, 1, _R), lambda i: (i, 0, 0)),
            pl.BlockSpec((_H, _H), fixed),
            pl.BlockSpec((_H, _H), fixed),
            pl.BlockSpec((1, _H), fixed),
            pl.BlockSpec((_H, _H), fixed),
            pl.BlockSpec((_H, _H), fixed),
            pl.BlockSpec((1, _H), fixed),
            pl.BlockSpec((_H, 1), fixed),
            pl.BlockSpec((1, 1), fixed),
            pl.BlockSpec((_H, 1), fixed),
            pl.BlockSpec((1, 1), fixed),
        ],
        out_specs=[pl.BlockSpec((_G, 1), fixed), pl.BlockSpec((_G, 1), fixed)],
        out_shape=[jax.ShapeDtypeStruct((_G, 1), jnp.float32),
                   jax.ShapeDtypeStruct((_G, 1), jnp.float32)],
        scratch_shapes=[
            pltpu.VMEM((_G, _H), jnp.float32),
            pltpu.VMEM((_G, _H), jnp.float32),
            pltpu.VMEM((1, _G), jnp.float32),
        ],
    )(a0, a1, h1, c0, c1, batch3, wr2, wt2, b2, wr3, wt3, b3,
      wl1, bl1, wl2, bl2)


def kernel(x, edge_index, batch, W_rel1, b_rel1, W_root1, W_rel2, b_rel2,
           W_root2, W_rel3, b_rel3, W_root3, W_lin1, b_lin1, W_lin2, b_lin2):
    src = edge_index[0]
    dst = edge_index[1]
    xp = jnp.pad(x, ((0, 0), (0, _XP - x.shape[1])))
    wr1 = jnp.pad(W_rel1, ((0, _XP - W_rel1.shape[0]), (0, 0)))
    wt1 = jnp.pad(W_root1, ((0, _XP - W_root1.shape[0]), (0, 0)))
    z16 = jnp.zeros((_NP, _XP), jnp.float32)
    zc = jnp.zeros((_N * _G,), jnp.float32)
    z128 = jnp.zeros((_NP, _H), jnp.float32)
    ones = jnp.ones((_K1,), jnp.float32)

    agg1p, cntp = _sc1_call(xp, src, dst, batch, z16, zc, ones)
    h1 = _tc1_call(agg1p[0, :_N], agg1p[1, :_N], xp, wr1, wt1, b_rel1.reshape(1, _H))
    agg2p = _sc2_call(h1, src, dst, z128)
    cntp = cntp.reshape(_NC, _N, _G)
    c0 = cntp[0]
    c1 = cntp[1]
    batch3 = batch.reshape(_NCH, 1, _R)
    x1, x2 = _tc2_call(agg2p[0, :_N], agg2p[1, :_N], h1, c0, c1, batch3,
                       W_rel2, W_root2, b_rel2.reshape(1, _H),
                       W_rel3, W_root3, b_rel3.reshape(1, _H),
                       W_lin1, b_lin1.reshape(1, 1),
                       W_lin2, b_lin2.reshape(1, 1))
    return (x1, x2)
